# Initial kernel scaffold; baseline (speedup 1.0000x reference)
import jax
import jax.numpy as jnp
from jax.experimental import pallas as pl


def kernel(W, b, action_indecies, action_n_obj, action_types):
    raise NotImplementedError("write your pallas kernel here")


# speedup vs baseline: 3.3164x; 3.3164x over previous
import functools

import jax
import jax.numpy as jnp
from jax import lax
from jax.experimental import pallas as pl
from jax.experimental.pallas import tpu as pltpu
from jax.experimental.pallas import tpu_sc as plsc

_LANES = 16


def _build_sc_call(B, E, N, O):
    info = plsc.get_sparse_core_info()
    NC, NS = info.num_cores, info.num_subcores
    NW = NC * NS
    assert B % NW == 0
    BPW = B // NW

    mesh = plsc.VectorSubcoreMesh(core_axis_name="c", subcore_axis_name="s")

    @functools.partial(
        pl.kernel,
        mesh=mesh,
        compiler_params=pltpu.CompilerParams(use_tc_tiling_on_sc=False),
        out_type=jax.ShapeDtypeStruct((B, _LANES), jnp.float32),
        scratch_types=[
            pltpu.VMEM((BPW,), jnp.int32),
            pltpu.VMEM((BPW,), jnp.int32),
            pltpu.VMEM((BPW,), jnp.int32),
            pltpu.VMEM((BPW, _LANES), jnp.float32),
            pltpu.VMEM((BPW, _LANES), jnp.float32),
            pltpu.VMEM((BPW, _LANES), jnp.float32),
            pltpu.SemaphoreType.DMA,
            pltpu.SemaphoreType.DMA,
        ],
    )
    def sc_call(w2_hbm, b2_hbm, idx_hbm, typ_hbm, out_hbm,
                idx_v, typ_v, cidx_v, w_v, bias_v, out_v, sem_w, sem_b):
        wid = lax.axis_index("s") * NC + lax.axis_index("c")
        base = wid * BPW
        pltpu.sync_copy(idx_hbm.at[pl.ds(base, BPW)], idx_v)
        pltpu.sync_copy(typ_hbm.at[pl.ds(base, BPW)], typ_v)

        for j in range(BPW // _LANES):
            sl = pl.ds(j * _LANES, _LANES)
            cidx_v[sl] = typ_v[sl] * N + idx_v[sl]

        cw = pltpu.async_copy(w2_hbm.at[cidx_v], w_v, sem_w)
        cb = pltpu.async_copy(b2_hbm.at[typ_v], bias_v, sem_b)
        cw.wait()
        cb.wait()

        def row(r, carry):
            x = w_v[r] + bias_v[r]
            a = jnp.abs(x)
            e = jnp.exp(a * (-2.0))
            t = (1.0 - e) / (1.0 + e)
            out_v[r] = jnp.sign(x) * t
            return carry

        lax.fori_loop(0, BPW, row, 0)

        pltpu.sync_copy(out_v, out_hbm.at[pl.ds(base, BPW)])

    return sc_call


def kernel(W, b, action_indecies, action_n_obj, action_types):
    del action_n_obj
    E, O, N = W.shape
    B = action_indecies.shape[0]
    w2 = jnp.transpose(W, (0, 2, 1)).reshape(E * N, O)
    w2 = jnp.pad(w2, ((0, 0), (0, _LANES - O)))
    b2 = jnp.pad(b, ((0, 0), (0, _LANES - O)))
    idx = action_indecies.astype(jnp.int32)
    typ = action_types.astype(jnp.int32)
    out_pad = _build_sc_call(B, E, N, O)(w2, b2, idx, typ)
    return out_pad[:, :O]


# speedup vs baseline: 3.3414x; 1.0075x over previous
import functools

import jax
import jax.numpy as jnp
from jax import lax
from jax.experimental import pallas as pl
from jax.experimental.pallas import tpu as pltpu
from jax.experimental.pallas import tpu_sc as plsc

_LANES = 16


def _build_sc_call(B, E, N, O):
    info = plsc.get_sparse_core_info()
    NC, NS = info.num_cores, info.num_subcores
    NW = NC * NS
    assert B % NW == 0
    BPW = B // NW

    mesh = plsc.VectorSubcoreMesh(core_axis_name="c", subcore_axis_name="s")

    @functools.partial(
        pl.kernel,
        mesh=mesh,
        compiler_params=pltpu.CompilerParams(use_tc_tiling_on_sc=False),
        out_type=jax.ShapeDtypeStruct((B, _LANES), jnp.float32),
        scratch_types=[
            pltpu.VMEM((BPW,), jnp.int32),
            pltpu.VMEM((BPW,), jnp.int32),
            pltpu.VMEM((BPW,), jnp.int32),
            pltpu.VMEM((BPW, _LANES), jnp.float32),
            pltpu.VMEM((BPW, _LANES), jnp.float32),
            pltpu.VMEM((BPW, _LANES), jnp.float32),
            pltpu.SemaphoreType.DMA,
            pltpu.SemaphoreType.DMA,
        ],
    )
    def sc_call(w2_hbm, b2_hbm, idx_hbm, typ_hbm, out_hbm,
                idx_v, typ_v, cidx_v, w_v, bias_v, out_v, sem_w, sem_b):
        wid = lax.axis_index("s") * NC + lax.axis_index("c")
        base = wid * BPW
        pltpu.sync_copy(idx_hbm.at[pl.ds(base, BPW)], idx_v)
        pltpu.sync_copy(typ_hbm.at[pl.ds(base, BPW)], typ_v)

        for j in range(BPW // _LANES):
            sl = pl.ds(j * _LANES, _LANES)
            cidx_v[sl] = typ_v[sl] * N + idx_v[sl]

        cw = pltpu.async_copy(w2_hbm.at[cidx_v], w_v, sem_w)
        cb = pltpu.async_copy(b2_hbm.at[typ_v], bias_v, sem_b)
        cw.wait()
        cb.wait()

        @plsc.parallel_loop(0, BPW, step=1, unroll=8)
        def row(r):
            x = w_v[r] + bias_v[r]
            e = jnp.exp(x * 2.0)
            out_v[r] = 1.0 - 2.0 / (e + 1.0)

        pltpu.sync_copy(out_v, out_hbm.at[pl.ds(base, BPW)])

    return sc_call


def kernel(W, b, action_indecies, action_n_obj, action_types):
    del action_n_obj
    E, O, N = W.shape
    B = action_indecies.shape[0]
    w2 = jnp.transpose(W, (0, 2, 1)).reshape(E * N, O)
    w2 = jnp.pad(w2, ((0, 0), (0, _LANES - O)))
    b2 = jnp.pad(b, ((0, 0), (0, _LANES - O)))
    idx = action_indecies.astype(jnp.int32)
    typ = action_types.astype(jnp.int32)
    out_pad = _build_sc_call(B, E, N, O)(w2, b2, idx, typ)
    return out_pad[:, :O]


# speedup vs baseline: 5.5847x; 1.6714x over previous
import functools

import jax
import jax.numpy as jnp
from jax import lax
from jax.experimental import pallas as pl
from jax.experimental.pallas import tpu as pltpu
from jax.experimental.pallas import tpu_sc as plsc

_LANES = 16


def _build_sc_call(B, E, N, O):
    info = plsc.get_sparse_core_info()
    NC, NS = info.num_cores, info.num_subcores
    NW = NC * NS
    assert B % NW == 0
    BPW = B // NW
    SPV = _LANES // O
    GPW = BPW // SPV

    mesh = plsc.VectorSubcoreMesh(core_axis_name="c", subcore_axis_name="s")

    @functools.partial(
        pl.kernel,
        mesh=mesh,
        compiler_params=pltpu.CompilerParams(
            use_tc_tiling_on_sc=False, needs_layout_passes=False
        ),
        out_type=jax.ShapeDtypeStruct((B * O // _LANES, _LANES), jnp.float32),
        scratch_types=[
            pltpu.VMEM((E * O * N,), jnp.float32),
            pltpu.VMEM((E * O,), jnp.float32),
            pltpu.VMEM((BPW,), jnp.int32),
            pltpu.VMEM((BPW,), jnp.int32),
            pltpu.VMEM((GPW, _LANES), jnp.float32),
            pltpu.SemaphoreType.DMA,
            pltpu.SemaphoreType.DMA,
        ],
    )
    def sc_call(w_hbm, b_hbm, idx_hbm, typ_hbm, out_hbm,
                wtab_v, btab_v, idx_v, typ_v, out_v, sem_w, sem_b):
        wid = lax.axis_index("s") * NC + lax.axis_index("c")
        base = wid * BPW
        cw = pltpu.async_copy(w_hbm, wtab_v, sem_w)
        cb = pltpu.async_copy(b_hbm, btab_v, sem_b)
        pltpu.sync_copy(idx_hbm.at[pl.ds(base, BPW)], idx_v)
        pltpu.sync_copy(typ_hbm.at[pl.ds(base, BPW)], typ_v)

        lane = lax.broadcasted_iota(jnp.int32, (_LANES,), 0)
        l4 = lane // SPV
        olane = lane - l4 * SPV
        ol_n = olane * N

        cb.wait()
        cw.wait()

        @plsc.parallel_loop(0, GPW, step=1, unroll=8)
        def group(r):
            sidx = l4 + r * SPV
            t2 = plsc.load_gather(typ_v, [sidx])
            i2 = plsc.load_gather(idx_v, [sidx])
            widx = t2 * (O * N) + ol_n + i2
            wv = plsc.load_gather(wtab_v, [widx])
            bv = plsc.load_gather(btab_v, [t2 * O + olane])
            x = wv + bv
            e = jnp.exp(x * 2.0)
            out_v[r] = 1.0 - 2.0 / (e + 1.0)

        pltpu.sync_copy(out_v, out_hbm.at[pl.ds(wid * GPW, GPW)])

    return sc_call


def kernel(W, b, action_indecies, action_n_obj, action_types):
    del action_n_obj
    E, O, N = W.shape
    B = action_indecies.shape[0]
    idx = action_indecies.astype(jnp.int32)
    typ = action_types.astype(jnp.int32)
    out = _build_sc_call(B, E, N, O)(W.reshape(-1), b.reshape(-1), idx, typ)
    return out.reshape(B, O)
